# speedup vs baseline: 2.6803x; 2.6803x over previous
import functools

import jax
import jax.numpy as jnp
from jax import lax
from jax.experimental import pallas as pl
from jax.experimental.pallas import tpu as pltpu

HIDDEN = 2048
EXPERTS = 16
MIN_K = 2


def _gating_body(h_ref, w_ref, g_ref, t_ref, mask_ref, logits_ref):
    h = h_ref[...]
    w = w_ref[...]

    colnorm = jnp.sqrt(jnp.sum(w * w, axis=0, keepdims=True))
    wn = w / jnp.maximum(colnorm, 1e-12)

    rownorm = jnp.sqrt(jnp.sum(h * h, axis=1, keepdims=True))
    hn = h / jnp.maximum(rownorm, 1e-12)
    logits = jnp.dot(hn, wn, preferred_element_type=jnp.float32)

    s = jax.nn.sigmoid(t_ref[0])
    scaled = logits * s
    sg = g_ref[...] * s
    gated = jnp.maximum(scaled - sg, 0.0)
    mask = (gated > 0.0).astype(jnp.float32)
    inactive = jnp.sum(mask, axis=1, keepdims=True) == 0.0

    blk = logits.shape[0]
    iota = lax.broadcasted_iota(jnp.int32, (blk, EXPERTS), 1)
    m1 = jnp.max(logits, axis=1, keepdims=True)
    i1 = jnp.min(jnp.where(logits == m1, iota, EXPERTS), axis=1, keepdims=True)
    neg = jnp.float32(-jnp.inf)
    l2 = jnp.where(iota == i1, neg, logits)
    m2 = jnp.max(l2, axis=1, keepdims=True)
    i2 = jnp.min(jnp.where(l2 == m2, iota, EXPERTS), axis=1, keepdims=True)
    fb = ((iota == i1) | (iota == i2)).astype(jnp.float32)

    mask_ref[...] = jnp.where(inactive, fb, mask)
    logits_ref[...] = logits


@functools.partial(jax.jit, static_argnames=("blk",))
def _gating(flat_h, sim_matrix, gates, temperature, blk):
    n = flat_h.shape[0]
    grid = (n // blk,)
    return pl.pallas_call(
        _gating_body,
        grid=grid,
        in_specs=[
            pl.BlockSpec((blk, HIDDEN), lambda i: (i, 0)),
            pl.BlockSpec((HIDDEN, EXPERTS), lambda i: (0, 0)),
            pl.BlockSpec((1, EXPERTS), lambda i: (0, 0)),
            pl.BlockSpec(memory_space=pltpu.SMEM),
        ],
        out_specs=[
            pl.BlockSpec((blk, EXPERTS), lambda i: (i, 0)),
            pl.BlockSpec((blk, EXPERTS), lambda i: (i, 0)),
        ],
        out_shape=[
            jax.ShapeDtypeStruct((n, EXPERTS), jnp.float32),
            jax.ShapeDtypeStruct((n, EXPERTS), jnp.float32),
        ],
    )(flat_h, sim_matrix, gates.reshape(1, EXPERTS),
      temperature.reshape(1).astype(jnp.float32))


def kernel(hidden_states, sim_matrix, gates, temperature):
    b, t, c = hidden_states.shape
    flat_h = hidden_states.reshape(b * t, c)
    mask, logits = _gating(flat_h, sim_matrix, gates, temperature, blk=512)
    return (mask, logits)


# speedup vs baseline: 3.1496x; 1.1751x over previous
import functools

import jax
import jax.numpy as jnp
from jax import lax
from jax.experimental import pallas as pl
from jax.experimental.pallas import tpu as pltpu

HIDDEN = 2048
EXPERTS = 16
MIN_K = 2


def _gating_body(h_ref, w_ref, g_ref, t_ref, mask_ref, logits_ref):
    h = h_ref[...]
    w = w_ref[...]

    colnorm = jnp.sqrt(jnp.sum(w * w, axis=0, keepdims=True))
    wn = w / jnp.maximum(colnorm, 1e-12)

    rownorm = jnp.sqrt(jnp.sum(h * h, axis=1, keepdims=True))
    hn = h / jnp.maximum(rownorm, 1e-12)
    logits = jnp.dot(hn, wn, preferred_element_type=jnp.float32)

    s = jax.nn.sigmoid(t_ref[0])
    scaled = logits * s
    sg = g_ref[...] * s
    gated = jnp.maximum(scaled - sg, 0.0)
    mask = (gated > 0.0).astype(jnp.float32)
    inactive = jnp.sum(mask, axis=1, keepdims=True) == 0.0

    blk = logits.shape[0]
    iota = lax.broadcasted_iota(jnp.int32, (blk, EXPERTS), 1)
    m1 = jnp.max(logits, axis=1, keepdims=True)
    i1 = jnp.min(jnp.where(logits == m1, iota, EXPERTS), axis=1, keepdims=True)
    neg = jnp.float32(-jnp.inf)
    l2 = jnp.where(iota == i1, neg, logits)
    m2 = jnp.max(l2, axis=1, keepdims=True)
    i2 = jnp.min(jnp.where(l2 == m2, iota, EXPERTS), axis=1, keepdims=True)
    fb = ((iota == i1) | (iota == i2)).astype(jnp.float32)

    mask_ref[...] = jnp.where(inactive, fb, mask)
    logits_ref[...] = logits


@functools.partial(jax.jit, static_argnames=("blk",))
def _gating(flat_h, sim_matrix, gates, temperature, blk):
    n = flat_h.shape[0]
    grid = (n // blk,)
    return pl.pallas_call(
        _gating_body,
        grid=grid,
        in_specs=[
            pl.BlockSpec((blk, HIDDEN), lambda i: (i, 0)),
            pl.BlockSpec((HIDDEN, EXPERTS), lambda i: (0, 0)),
            pl.BlockSpec((1, EXPERTS), lambda i: (0, 0)),
            pl.BlockSpec(memory_space=pltpu.SMEM),
        ],
        out_specs=[
            pl.BlockSpec((blk, EXPERTS), lambda i: (i, 0)),
            pl.BlockSpec((blk, EXPERTS), lambda i: (i, 0)),
        ],
        out_shape=[
            jax.ShapeDtypeStruct((n, EXPERTS), jnp.float32),
            jax.ShapeDtypeStruct((n, EXPERTS), jnp.float32),
        ],
    )(flat_h, sim_matrix, gates.reshape(1, EXPERTS),
      temperature.reshape(1).astype(jnp.float32))


def kernel(hidden_states, sim_matrix, gates, temperature):
    b, t, c = hidden_states.shape
    flat_h = hidden_states.reshape(b * t, c)
    mask, logits = _gating(flat_h, sim_matrix, gates, temperature, blk=1024)
    return (mask, logits)


# speedup vs baseline: 3.3287x; 1.0569x over previous
import functools

import jax
import jax.numpy as jnp
from jax import lax
from jax.experimental import pallas as pl
from jax.experimental.pallas import tpu as pltpu

HIDDEN = 2048
EXPERTS = 16
MIN_K = 2


def _gating_body(h_ref, w_ref, g_ref, t_ref, mask_ref, logits_ref):
    h = h_ref[...]
    w = w_ref[...]

    colnorm = jnp.sqrt(jnp.sum(w * w, axis=0, keepdims=True))
    wn = w / jnp.maximum(colnorm, 1e-12)

    rownorm = jnp.sqrt(jnp.sum(h * h, axis=1, keepdims=True))
    hn = h / jnp.maximum(rownorm, 1e-12)
    logits = jnp.dot(hn, wn, preferred_element_type=jnp.float32)

    s = jax.nn.sigmoid(t_ref[0])
    scaled = logits * s
    sg = g_ref[...] * s
    gated = jnp.maximum(scaled - sg, 0.0)
    mask = (gated > 0.0).astype(jnp.float32)
    inactive = jnp.sum(mask, axis=1, keepdims=True) == 0.0

    blk = logits.shape[0]
    iota = lax.broadcasted_iota(jnp.int32, (blk, EXPERTS), 1)
    m1 = jnp.max(logits, axis=1, keepdims=True)
    i1 = jnp.min(jnp.where(logits == m1, iota, EXPERTS), axis=1, keepdims=True)
    neg = jnp.float32(-jnp.inf)
    l2 = jnp.where(iota == i1, neg, logits)
    m2 = jnp.max(l2, axis=1, keepdims=True)
    i2 = jnp.min(jnp.where(l2 == m2, iota, EXPERTS), axis=1, keepdims=True)
    fb = ((iota == i1) | (iota == i2)).astype(jnp.float32)

    mask_ref[...] = jnp.where(inactive, fb, mask)
    logits_ref[...] = logits


@functools.partial(jax.jit, static_argnames=("blk",))
def _gating(flat_h, sim_matrix, gates, temperature, blk):
    n = flat_h.shape[0]
    grid = (n // blk,)
    return pl.pallas_call(
        _gating_body,
        grid=grid,
        in_specs=[
            pl.BlockSpec((blk, HIDDEN), lambda i: (i, 0)),
            pl.BlockSpec((HIDDEN, EXPERTS), lambda i: (0, 0)),
            pl.BlockSpec((1, EXPERTS), lambda i: (0, 0)),
            pl.BlockSpec(memory_space=pltpu.SMEM),
        ],
        out_specs=[
            pl.BlockSpec((blk, EXPERTS), lambda i: (i, 0)),
            pl.BlockSpec((blk, EXPERTS), lambda i: (i, 0)),
        ],
        out_shape=[
            jax.ShapeDtypeStruct((n, EXPERTS), jnp.float32),
            jax.ShapeDtypeStruct((n, EXPERTS), jnp.float32),
        ],
    )(flat_h, sim_matrix, gates.reshape(1, EXPERTS),
      temperature.reshape(1).astype(jnp.float32))


def kernel(hidden_states, sim_matrix, gates, temperature):
    b, t, c = hidden_states.shape
    flat_h = hidden_states.reshape(b * t, c)
    mask, logits = _gating(flat_h, sim_matrix, gates, temperature, blk=2048)
    return (mask, logits)
